# baseline (speedup 1.0000x reference)
import functools

import jax
import jax.numpy as jnp
from jax import lax
from jax.experimental import pallas as pl
from jax.experimental.pallas import tpu as pltpu
from jax.experimental.pallas import tpu_sc as plsc

LANES = 16
DIM = 64
NVEC = DIM // LANES
N1 = 128
N2 = 80
NPAD = N1 + N2
NCHUNK = NPAD // LANES
CSLACK = NPAD + LANES


def _build_sc_call(B):
    info = plsc.get_sparse_core_info()
    NC, NS = info.num_cores, info.num_subcores
    NW = NC * NS
    assert B % NW == 0
    bpw = B // NW

    mesh = plsc.VectorSubcoreMesh(core_axis_name="c", subcore_axis_name="s")

    @functools.partial(
        pl.kernel,
        out_type=jax.ShapeDtypeStruct((B, DIM), jnp.float32),
        mesh=mesh,
        compiler_params=pltpu.CompilerParams(use_tc_tiling_on_sc=False,
                                             needs_layout_passes=False),
        scratch_types=[
            pltpu.VMEM((bpw * NPAD,), jnp.int32),
            pltpu.VMEM((bpw * NPAD,), jnp.int32),
            pltpu.VMEM((2, CSLACK), jnp.int32),
            pltpu.VMEM((2, NPAD, DIM), jnp.float32),
            pltpu.VMEM((2, LANES), jnp.float32),
            pltpu.VMEM((2, LANES), jnp.int32),
            pltpu.VMEM((bpw, DIM), jnp.float32),
            pltpu.SemaphoreType.DMA((2,)),
            pltpu.SemaphoreType.DMA((2,)),
        ],
    )
    def sc_embed(ids_hbm, mask_hbm, table_hbm, out_hbm,
                 ids_v, mask_v, cidx, rows, denom_v, cnt_v, out_v,
                 sem1, sem2):
        wid = lax.axis_index("s") * NC + lax.axis_index("c")
        flat0 = wid * (bpw * NPAD)
        pltpu.sync_copy(ids_hbm.at[pl.ds(flat0, bpw * NPAD)], ids_v)
        pltpu.sync_copy(mask_hbm.at[pl.ds(flat0, bpw * NPAD)], mask_v)
        for s in range(2):
            for k in range(CSLACK // LANES):
                cidx[s, pl.ds(k * LANES, LANES)] = jnp.zeros(
                    (LANES,), jnp.int32)

        def prep(b, slot):
            base = b * NPAD
            off = jnp.int32(0)
            for j in range(NCHUNK):
                m_c = mask_v[pl.ds(base + j * LANES, LANES)]
                ids_c = ids_v[pl.ds(base + j * LANES, LANES)]
                live = m_c > 0
                plsc.store_compressed(
                    cidx.at[slot, pl.ds(off, LANES)], ids_c, mask=live)
                off = off + plsc.all_reduce_population_count(live)[0]
            total = off.astype(jnp.float32)
            denom_v[slot, :] = jnp.maximum(
                jnp.broadcast_to(total, (LANES,)), 1.0)
            cnt_v[slot, :] = jnp.broadcast_to(off, (LANES,))
            pltpu.async_copy(table_hbm.at[cidx.at[slot, pl.ds(0, N1)]],
                             rows.at[slot, pl.ds(0, N1)], sem1.at[slot])

            @pl.when(off > N1)
            def _():
                pltpu.async_copy(table_hbm.at[cidx.at[slot, pl.ds(N1, N2)]],
                                 rows.at[slot, pl.ds(N1, N2)], sem2.at[slot])

        def accum(b, slot):
            d = denom_v[slot, :]
            cnt = cnt_v[slot, :][0]
            pltpu.make_async_copy(table_hbm.at[cidx.at[slot, pl.ds(0, N1)]],
                                  rows.at[slot, pl.ds(0, N1)],
                                  sem1.at[slot]).wait()

            @pl.when(cnt > N1)
            def _():
                pltpu.make_async_copy(
                    table_hbm.at[cidx.at[slot, pl.ds(N1, N2)]],
                    rows.at[slot, pl.ds(N1, N2)], sem2.at[slot]).wait()

            full = lax.shift_right_logical(cnt, 4)
            rem = cnt & (LANES - 1)

            def body(g, acc):
                i0 = g * LANES
                for r in range(LANES):
                    acc = tuple(
                        acc[j] + rows[slot, i0 + r, pl.ds(j * LANES, LANES)]
                        for j in range(NVEC))
                return acc

            zeros = tuple(jnp.zeros((LANES,), jnp.float32)
                          for _ in range(NVEC))
            acc = lax.fori_loop(0, full, body, zeros)
            i0 = full * LANES
            for r in range(LANES):
                m = jnp.broadcast_to(
                    jnp.where(r < rem, 1.0, 0.0).astype(jnp.float32),
                    (LANES,))
                acc = tuple(
                    acc[j] + rows[slot, i0 + r, pl.ds(j * LANES, LANES)] * m
                    for j in range(NVEC))
            for j in range(NVEC):
                out_v[b, pl.ds(j * LANES, LANES)] = acc[j] / d

        prep(0, 0)

        def pair_body(k, _):
            prep(2 * k + 1, 1)
            accum(2 * k, 0)

            @pl.when(k < bpw // 2 - 1)
            def _():
                prep(2 * k + 2, 0)

            accum(2 * k + 1, 1)
            return _

        lax.fori_loop(0, bpw // 2, pair_body, None)
        pltpu.sync_copy(out_v, out_hbm.at[pl.ds(wid * bpw, bpw)])

    return sc_embed


def kernel(input_ids, attention_mask, table):
    B, L = input_ids.shape
    V = table.shape[0]
    pad = NPAD - L
    ids_flat = jnp.pad(input_ids.astype(jnp.int32),
                       ((0, 0), (0, pad))).reshape(-1)
    mask_flat = jnp.pad(attention_mask.astype(jnp.int32),
                        ((0, 0), (0, pad))).reshape(-1)
    sc = _build_sc_call(B)
    return sc(ids_flat, mask_flat, table)


# speedup vs baseline: 1.1844x; 1.1844x over previous
import functools

import jax
import jax.numpy as jnp
from jax import lax
from jax.experimental import pallas as pl
from jax.experimental.pallas import tpu as pltpu
from jax.experimental.pallas import tpu_sc as plsc

LANES = 16
DIM = 64
DH = 32
N1 = 128
N2 = 80
NPAD = N1 + N2
NCHUNK = NPAD // LANES


def _build_sc_call(B, d):
    info = plsc.get_sparse_core_info()
    NC, NS = info.num_cores, info.num_subcores
    NW = NC * NS
    assert B % NW == 0
    bpw = B // NW

    mesh = plsc.VectorSubcoreMesh(core_axis_name="c", subcore_axis_name="s")

    @functools.partial(
        pl.kernel,
        out_type=jax.ShapeDtypeStruct((B, d), jnp.float32),
        mesh=mesh,
        compiler_params=pltpu.CompilerParams(use_tc_tiling_on_sc=False,
                                             needs_layout_passes=False),
        scratch_types=[
            pltpu.VMEM((bpw * NPAD,), jnp.int32),
            pltpu.VMEM((bpw * NPAD,), jnp.int32),
            pltpu.VMEM((2, N1, d), jnp.float32),
            pltpu.VMEM((2, N2, d), jnp.float32),
            pltpu.VMEM((2, LANES), jnp.float32),
            pltpu.VMEM((bpw, d), jnp.float32),
            pltpu.SemaphoreType.DMA((2,)),
            pltpu.SemaphoreType.DMA((2,)),
        ],
    )
    def sc_embed(ids_hbm, mask_hbm, table_hbm, out_hbm,
                 ids_v, mask_v, rows1, rows2, denom_v, out_v, sem1, sem2):
        wid = lax.axis_index("s") * NC + lax.axis_index("c")
        flat0 = wid * (bpw * NPAD)
        pltpu.sync_copy(ids_hbm.at[pl.ds(flat0, bpw * NPAD)], ids_v)
        pltpu.sync_copy(mask_hbm.at[pl.ds(flat0, bpw * NPAD)], mask_v)

        def prep(b, slot):
            base = b * NPAD
            cnt = jnp.zeros((LANES,), jnp.int32)
            for j in range(NCHUNK):
                cnt = cnt + mask_v[pl.ds(base + j * LANES, LANES)]
            total = jnp.sum(cnt).astype(jnp.float32)
            denom_v[slot, :] = jnp.maximum(
                jnp.broadcast_to(total, (LANES,)), 1.0)
            pltpu.async_copy(table_hbm.at[ids_v.at[pl.ds(base, N1)]],
                             rows1.at[slot], sem1.at[slot])
            pltpu.async_copy(table_hbm.at[ids_v.at[pl.ds(base + N1, N2)]],
                             rows2.at[slot], sem2.at[slot])

        def accum(b, slot):
            base = b * NPAD
            pltpu.make_async_copy(table_hbm.at[ids_v.at[pl.ds(base, N1)]],
                                  rows1.at[slot], sem1.at[slot]).wait()
            pltpu.make_async_copy(
                table_hbm.at[ids_v.at[pl.ds(base + N1, N2)]],
                rows2.at[slot], sem2.at[slot]).wait()

            def group_body(rows_ref, off):
                def body(g, acc):
                    mf = mask_v[pl.ds(base + off + g * LANES, LANES)].astype(
                        jnp.float32)
                    i0 = g * LANES
                    for r in range(LANES):
                        m = jnp.broadcast_to(mf[r], (LANES,))
                        acc = tuple(
                            acc[j]
                            + rows_ref[slot, i0 + r, pl.ds(j * LANES, LANES)]
                            * m
                            for j in range(d // LANES))
                    return acc
                return body

            zeros = tuple(jnp.zeros((LANES,), jnp.float32)
                          for _ in range(d // LANES))
            acc = lax.fori_loop(0, N1 // LANES, group_body(rows1, 0), zeros)
            acc = lax.fori_loop(0, N2 // LANES, group_body(rows2, N1), acc)
            dn = denom_v[slot, :]
            for j in range(d // LANES):
                out_v[b, pl.ds(j * LANES, LANES)] = acc[j] / dn

        prep(0, 0)

        def pair_body(k, _):
            prep(2 * k + 1, 1)
            accum(2 * k, 0)

            @pl.when(k < bpw // 2 - 1)
            def _():
                prep(2 * k + 2, 0)

            accum(2 * k + 1, 1)
            return _

        lax.fori_loop(0, bpw // 2, pair_body, None)
        pltpu.sync_copy(out_v, out_hbm.at[pl.ds(wid * bpw, bpw)])

    return sc_embed


def kernel(input_ids, attention_mask, table):
    B, L = input_ids.shape
    V = table.shape[0]
    pad = NPAD - L
    dummy = (jnp.arange(B, dtype=jnp.int32)[:, None] * pad
             + jnp.arange(pad, dtype=jnp.int32)[None, :]) % V
    ids_flat = jnp.concatenate(
        [input_ids.astype(jnp.int32), dummy], axis=1).reshape(-1)
    mask_flat = jnp.pad(attention_mask.astype(jnp.int32),
                        ((0, 0), (0, pad))).reshape(-1)
    sc = _build_sc_call(B)
    return sc(ids_flat, mask_flat, table)


# speedup vs baseline: 2.4554x; 2.0730x over previous
import functools

import jax
import jax.numpy as jnp
from jax import lax
from jax.experimental import pallas as pl
from jax.experimental.pallas import tpu as pltpu
from jax.experimental.pallas import tpu_sc as plsc

LANES = 16
DIM = 64
N1 = 128
N2 = 80
NPAD = N1 + N2
NCHUNK = NPAD // LANES


def _build_sc_call(B):
    info = plsc.get_sparse_core_info()
    NC, NS = info.num_cores, info.num_subcores
    NW = NC * NS
    assert B % NW == 0
    bpw = B // NW

    mesh = plsc.VectorSubcoreMesh(core_axis_name="c", subcore_axis_name="s")

    @functools.partial(
        pl.kernel,
        out_type=jax.ShapeDtypeStruct((B, DIM), jnp.float32),
        mesh=mesh,
        compiler_params=pltpu.CompilerParams(use_tc_tiling_on_sc=False,
                                             needs_layout_passes=False),
        scratch_types=[
            pltpu.VMEM((bpw * NPAD,), jnp.int32),
            pltpu.VMEM((bpw * NPAD,), jnp.int32),
            pltpu.VMEM((2, N1, DIM), jnp.float32),
            pltpu.VMEM((2, N2, DIM), jnp.float32),
            pltpu.VMEM((2, LANES), jnp.float32),
            pltpu.VMEM((bpw, DIM), jnp.float32),
            pltpu.SemaphoreType.DMA((2,)),
            pltpu.SemaphoreType.DMA((2,)),
        ],
    )
    def sc_embed(ids_hbm, mask_hbm, table_hbm, out_hbm,
                 ids_v, mask_v, rows1, rows2, denom_v, out_v, sem1, sem2):
        wid = lax.axis_index("s") * NC + lax.axis_index("c")
        flat0 = wid * (bpw * NPAD)
        pltpu.sync_copy(ids_hbm.at[pl.ds(flat0, bpw * NPAD)], ids_v)
        pltpu.sync_copy(mask_hbm.at[pl.ds(flat0, bpw * NPAD)], mask_v)

        def prep(b, slot):
            base = b * NPAD
            cnt = jnp.zeros((LANES,), jnp.int32)
            for j in range(NCHUNK):
                cnt = cnt + mask_v[pl.ds(base + j * LANES, LANES)]
            total = jnp.sum(cnt).astype(jnp.float32)
            denom_v[slot, :] = jnp.maximum(
                jnp.broadcast_to(total, (LANES,)), 1.0)
            pltpu.async_copy(table_hbm.at[ids_v.at[pl.ds(base, N1)]],
                             rows1.at[slot], sem1.at[slot])
            pltpu.async_copy(table_hbm.at[ids_v.at[pl.ds(base + N1, N2)]],
                             rows2.at[slot], sem2.at[slot])

        def accum(b, slot):
            base = b * NPAD
            pltpu.make_async_copy(table_hbm.at[ids_v.at[pl.ds(base, N1)]],
                                  rows1.at[slot], sem1.at[slot]).wait()
            pltpu.make_async_copy(
                table_hbm.at[ids_v.at[pl.ds(base + N1, N2)]],
                rows2.at[slot], sem2.at[slot]).wait()

            def group_body(rows_ref, off):
                def body(g, acc):
                    mf = mask_v[pl.ds(base + off + g * LANES, LANES)].astype(
                        jnp.float32)
                    i0 = g * LANES
                    for r in range(LANES):
                        m = jnp.broadcast_to(mf[r], (LANES,))
                        acc = tuple(
                            acc[j]
                            + rows_ref[slot, i0 + r, pl.ds(j * LANES, LANES)]
                            * m
                            for j in range(DIM // LANES))
                    return acc
                return body

            zeros = tuple(jnp.zeros((LANES,), jnp.float32)
                          for _ in range(DIM // LANES))
            acc = lax.fori_loop(0, N1 // LANES, group_body(rows1, 0), zeros)
            acc = lax.fori_loop(0, N2 // LANES, group_body(rows2, N1), acc)
            d = denom_v[slot, :]
            for j in range(DIM // LANES):
                out_v[b, pl.ds(j * LANES, LANES)] = acc[j] / d

        prep(0, 0)

        def pair_body(k, _):
            prep(2 * k + 1, 1)
            accum(2 * k, 0)

            @pl.when(k < bpw // 2 - 1)
            def _():
                prep(2 * k + 2, 0)

            accum(2 * k + 1, 1)
            return _

        lax.fori_loop(0, bpw // 2, pair_body, None)
        pltpu.sync_copy(out_v, out_hbm.at[pl.ds(wid * bpw, bpw)])

    return sc_embed


def kernel(input_ids, attention_mask, table):
    B, L = input_ids.shape
    V = table.shape[0]
    pad = NPAD - L
    dummy = (jnp.arange(B, dtype=jnp.int32)[:, None] * pad
             + jnp.arange(pad, dtype=jnp.int32)[None, :]) % V
    ids_flat = jnp.concatenate(
        [input_ids.astype(jnp.int32), dummy], axis=1).reshape(-1)
    mask_flat = jnp.pad(attention_mask.astype(jnp.int32),
                        ((0, 0), (0, pad))).reshape(-1)
    sc = _build_sc_call(B)
    return sc(ids_flat, mask_flat, table)
